# baseline (speedup 1.0000x reference)
import functools

import jax
import jax.numpy as jnp
import numpy as np
from jax import lax
from jax.experimental import pallas as pl
from jax.experimental.pallas import tpu as pltpu
from jax.experimental.pallas import tpu_sc as plsc

_CHARGE = 18.222615
_N_ATOMS = 10000
_N_VDW = 320000

_NC = 2
_NS = 16
_L = 16
_NW = _NC * _NS
_EPW = _N_VDW // _NW
_SC_ITERS = _EPW // _L


def _sc_body(idx0_hbm, idx1_hbm, tbl_hbm,
             s6_hbm, e_hbm, q_hbm,
             tbl_v, i0_v, i1_v, s6_v, e_v, q_v,
             tbl_sh, sem, bsem):
    sid = lax.axis_index("s")
    wid = sid * _NC + lax.axis_index("c")
    base = wid * _EPW
    cps = [
        pltpu.async_copy(idx0_hbm.at[pl.ds(base, _EPW)], i0_v, sem),
        pltpu.async_copy(idx1_hbm.at[pl.ds(base, _EPW)], i1_v, sem),
    ]
    @pl.when(sid == 0)
    def _():
        pltpu.async_copy(tbl_hbm, tbl_sh, bsem).wait()

    plsc.subcore_barrier()
    cps.append(pltpu.async_copy(tbl_sh, tbl_v, sem))
    for cp in cps:
        cp.wait()

    def gather6(i0, i1):
        ie0 = i0 + _N_ATOMS
        ie1 = i1 + _N_ATOMS
        ic0 = ie0 + _N_ATOMS
        ic1 = ie1 + _N_ATOMS
        return (plsc.load_gather(tbl_v, [i0]), plsc.load_gather(tbl_v, [i1]),
                plsc.load_gather(tbl_v, [ie0]), plsc.load_gather(tbl_v, [ie1]),
                plsc.load_gather(tbl_v, [ic0]), plsc.load_gather(tbl_v, [ic1]))

    def emit(off, g):
        s1, s2, e1, e2, c1, c2 = g
        sg = s1 + s2
        sq = sg * sg
        s6_v[pl.ds(off, _L)] = sq * sq * sq
        e_v[pl.ds(off, _L)] = e1 * e2
        q_v[pl.ds(off, _L)] = c1 * c2

    def run_span(gs, n):
        g0 = gather6(i0_v[pl.ds(gs * _L, _L)], i1_v[pl.ds(gs * _L, _L)])
        nx = (gs + 1) * _L
        carry0 = (i0_v[pl.ds(nx, _L)], i1_v[pl.ds(nx, _L)]) + g0

        def body(i, carry):
            i0n, i1n = carry[0], carry[1]
            g = carry[2:]
            gn = gather6(i0n, i1n)
            off2 = jnp.minimum(gs + i + 2, _SC_ITERS - 1) * _L
            i0nn = i0_v[pl.ds(off2, _L)]
            i1nn = i1_v[pl.ds(off2, _L)]
            emit((gs + i) * _L, g)
            return (i0nn, i1nn) + gn

        last = lax.fori_loop(0, n - 1, body, carry0, unroll=4)
        emit((gs + n - 1) * _L, last[2:])

    _H1 = 313
    _E1 = _H1 * _L
    run_span(0, _H1)
    ocps = [
        pltpu.async_copy(s6_v.at[pl.ds(0, _E1)],
                         s6_hbm.at[pl.ds(base, _E1)], sem),
        pltpu.async_copy(e_v.at[pl.ds(0, _E1)],
                         e_hbm.at[pl.ds(base, _E1)], sem),
        pltpu.async_copy(q_v.at[pl.ds(0, _E1)],
                         q_hbm.at[pl.ds(base, _E1)], sem),
    ]
    run_span(_H1, _SC_ITERS - _H1)
    rem = _EPW - _E1
    ocps += [
        pltpu.async_copy(s6_v.at[pl.ds(_E1, rem)],
                         s6_hbm.at[pl.ds(base + _E1, rem)], sem),
        pltpu.async_copy(e_v.at[pl.ds(_E1, rem)],
                         e_hbm.at[pl.ds(base + _E1, rem)], sem),
        pltpu.async_copy(q_v.at[pl.ds(_E1, rem)],
                         q_hbm.at[pl.ds(base + _E1, rem)], sem),
    ]
    for cp in ocps:
        cp.wait()


@functools.lru_cache(maxsize=None)
def _build_sc_gather():
    return functools.partial(
        pl.kernel,
        mesh=plsc.VectorSubcoreMesh(core_axis_name="c", subcore_axis_name="s"),
        compiler_params=pltpu.CompilerParams(needs_layout_passes=False),
        out_type=[jax.ShapeDtypeStruct((_N_VDW,), jnp.float32)] * 3,
        scratch_types=[
        pltpu.VMEM((3 * _N_ATOMS,), jnp.float32),
        pltpu.VMEM((_EPW,), jnp.int32),
        pltpu.VMEM((_EPW,), jnp.int32),
        pltpu.VMEM((_EPW,), jnp.float32),
        pltpu.VMEM((_EPW,), jnp.float32),
        pltpu.VMEM((_EPW,), jnp.float32),
        pltpu.VMEM_SHARED((3 * _N_ATOMS,), jnp.float32),
        pltpu.SemaphoreType.DMA,
        pltpu.SemaphoreType.DMA,
        ],
    )(_sc_body)


def _vdw_body(lv_ref, s6_ref, e_ref, q_ref, v14_ref, c14_ref, ev_ref, ec_ref):
    sl = pl.ds(pl.program_id(0) * _BV, _BV)
    qscale = _CHARGE * _CHARGE / 100.0
    lv = lv_ref[...]
    r = 1.0 / lv
    r2 = r * r
    r6 = r2 * r2 * r2
    t = s6_ref[sl][None, :] * r6
    em = (e_ref[sl] * v14_ref[sl] * 0.01)[None, :]
    qm = (q_ref[sl] * c14_ref[sl] * qscale)[None, :]
    ev_ref[...] = em * (t * t - 2.0 * t)
    ec_ref[...] = qm * r


def _small_body(lb_ref, pb_ref, ta_ref, pa_ref, sc_ref, pt_ref, ci_ref, pi_ref,
                eb_ref, ea_ref, et_ref, ei_ref):
    db = lb_ref[...] - pb_ref[1:2, :]
    eb_ref[...] = (pb_ref[0:1, :] * 100.0) * db * db

    da = ta_ref[...] - pa_ref[1:2, :] * np.float32(np.pi / 10.0)
    ea_ref[...] = (pa_ref[0:1, :] * 10.0) * da * da

    et_ref[...] = (pt_ref[0:1, :] * sc_ref[:, 1, :]
                   + pt_ref[1:2, :] * sc_ref[:, 3, :]
                   + pt_ref[2:3, :] * sc_ref[:, 5, :]
                   + pt_ref[3:4, :] * sc_ref[:, 7, :])

    ei_ref[...] = pi_ref[...] * (1.0 - ci_ref[...])


_G = 4
_BV = _N_VDW // _G


def _row_spec(b):
    return pl.BlockSpec((16, b), lambda i: (0, i))


def _vec_spec(b):
    del b
    return pl.BlockSpec((_N_VDW,), lambda i: (0,))


_vdw_call = pl.pallas_call(
    _vdw_body,
    grid=(_G,),
    in_specs=[
        _row_spec(_BV), _vec_spec(_BV), _vec_spec(_BV), _vec_spec(_BV),
        _vec_spec(_BV), _vec_spec(_BV),
    ],
    out_specs=[_row_spec(_BV), _row_spec(_BV)],
    out_shape=[
        jax.ShapeDtypeStruct((16, _N_VDW), jnp.float32),
        jax.ShapeDtypeStruct((16, _N_VDW), jnp.float32),
    ],
)

_small_call = pl.pallas_call(
    _small_body,
    out_shape=[
        jax.ShapeDtypeStruct((16, 10000), jnp.float32),
        jax.ShapeDtypeStruct((16, 20000), jnp.float32),
        jax.ShapeDtypeStruct((16, 30000), jnp.float32),
        jax.ShapeDtypeStruct((16, 5000), jnp.float32),
    ],
)


def kernel(length_bond, theta_angle, length_vdw, non_bonded, vdw14, charge14,
           sin_cos_n_theta_torsion, cos2_imptors, paras_bond, paras_angle,
           paras_vdw, paras_charge, paras_torsion, paras_imptors):
    f32 = jnp.float32
    nb = non_bonded.astype(jnp.int32)

    tbl = jnp.concatenate(
        [paras_vdw[:, 0], paras_vdw[:, 1], paras_charge.astype(f32)])
    s6, e, q = _build_sc_gather()(nb[0], nb[1], tbl)

    E_bond, E_angle, E_torsion, E_imptors = _small_call(
        length_bond, paras_bond.T,
        theta_angle, paras_angle.T,
        jnp.transpose(sin_cos_n_theta_torsion, (0, 2, 1)), paras_torsion.T,
        cos2_imptors, paras_imptors.T,
    )

    E_vdw, E_charge = _vdw_call(length_vdw, s6, e, q, vdw14, charge14)

    E_ub = jnp.zeros((length_vdw.shape[0], 1), dtype=length_vdw.dtype)
    return (E_bond, E_angle, E_ub, E_vdw, E_charge, E_torsion, E_imptors)


# speedup vs baseline: 1.0019x; 1.0019x over previous
import functools

import jax
import jax.numpy as jnp
import numpy as np
from jax import lax
from jax.experimental import pallas as pl
from jax.experimental.pallas import tpu as pltpu
from jax.experimental.pallas import tpu_sc as plsc

_CHARGE = 18.222615
_N_ATOMS = 10000
_N_VDW = 320000

_NC = 2
_NS = 16
_L = 16
_NW = _NC * _NS
_EPW = _N_VDW // _NW
_SC_ITERS = _EPW // _L


def _sc_body(idx0_hbm, idx1_hbm, tbl_hbm,
             s6_hbm, e_hbm, q_hbm,
             sig_v, eps_v, chg_v, i0_v, i1_v, s6_v, e_v, q_v,
             tbl_sh, sem, bsem):
    sid = lax.axis_index("s")
    wid = sid * _NC + lax.axis_index("c")
    base = wid * _EPW
    cps = [
        pltpu.async_copy(idx0_hbm.at[pl.ds(base, _EPW)], i0_v, sem),
        pltpu.async_copy(idx1_hbm.at[pl.ds(base, _EPW)], i1_v, sem),
    ]
    @pl.when(sid == 0)
    def _():
        pltpu.async_copy(tbl_hbm, tbl_sh, bsem).wait()

    plsc.subcore_barrier()
    cps += [
        pltpu.async_copy(tbl_sh.at[pl.ds(0, _N_ATOMS)], sig_v, sem),
        pltpu.async_copy(tbl_sh.at[pl.ds(_N_ATOMS, _N_ATOMS)], eps_v, sem),
        pltpu.async_copy(tbl_sh.at[pl.ds(2 * _N_ATOMS, _N_ATOMS)], chg_v, sem),
    ]
    for cp in cps:
        cp.wait()

    def gather6(i0, i1):
        return (plsc.load_gather(sig_v, [i0]), plsc.load_gather(sig_v, [i1]),
                plsc.load_gather(eps_v, [i0]), plsc.load_gather(eps_v, [i1]),
                plsc.load_gather(chg_v, [i0]), plsc.load_gather(chg_v, [i1]))

    def emit(off, g):
        s1, s2, e1, e2, c1, c2 = g
        sg = s1 + s2
        sq = sg * sg
        s6_v[pl.ds(off, _L)] = sq * sq * sq
        e_v[pl.ds(off, _L)] = e1 * e2
        q_v[pl.ds(off, _L)] = c1 * c2

    def run_span(gs, n):
        g0 = gather6(i0_v[pl.ds(gs * _L, _L)], i1_v[pl.ds(gs * _L, _L)])
        nx = (gs + 1) * _L
        carry0 = (i0_v[pl.ds(nx, _L)], i1_v[pl.ds(nx, _L)]) + g0

        def body(i, carry):
            i0n, i1n = carry[0], carry[1]
            g = carry[2:]
            gn = gather6(i0n, i1n)
            off2 = jnp.minimum(gs + i + 2, _SC_ITERS - 1) * _L
            i0nn = i0_v[pl.ds(off2, _L)]
            i1nn = i1_v[pl.ds(off2, _L)]
            emit((gs + i) * _L, g)
            return (i0nn, i1nn) + gn

        last = lax.fori_loop(0, n - 1, body, carry0, unroll=4)
        emit((gs + n - 1) * _L, last[2:])

    _H1 = 313
    _E1 = _H1 * _L
    run_span(0, _H1)
    ocps = [
        pltpu.async_copy(s6_v.at[pl.ds(0, _E1)],
                         s6_hbm.at[pl.ds(base, _E1)], sem),
        pltpu.async_copy(e_v.at[pl.ds(0, _E1)],
                         e_hbm.at[pl.ds(base, _E1)], sem),
        pltpu.async_copy(q_v.at[pl.ds(0, _E1)],
                         q_hbm.at[pl.ds(base, _E1)], sem),
    ]
    run_span(_H1, _SC_ITERS - _H1)
    rem = _EPW - _E1
    ocps += [
        pltpu.async_copy(s6_v.at[pl.ds(_E1, rem)],
                         s6_hbm.at[pl.ds(base + _E1, rem)], sem),
        pltpu.async_copy(e_v.at[pl.ds(_E1, rem)],
                         e_hbm.at[pl.ds(base + _E1, rem)], sem),
        pltpu.async_copy(q_v.at[pl.ds(_E1, rem)],
                         q_hbm.at[pl.ds(base + _E1, rem)], sem),
    ]
    for cp in ocps:
        cp.wait()


@functools.lru_cache(maxsize=None)
def _build_sc_gather():
    return functools.partial(
        pl.kernel,
        mesh=plsc.VectorSubcoreMesh(core_axis_name="c", subcore_axis_name="s"),
        compiler_params=pltpu.CompilerParams(needs_layout_passes=False),
        out_type=[jax.ShapeDtypeStruct((_N_VDW,), jnp.float32)] * 3,
        scratch_types=[
        pltpu.VMEM((_N_ATOMS,), jnp.float32),
        pltpu.VMEM((_N_ATOMS,), jnp.float32),
        pltpu.VMEM((_N_ATOMS,), jnp.float32),
        pltpu.VMEM((_EPW,), jnp.int32),
        pltpu.VMEM((_EPW,), jnp.int32),
        pltpu.VMEM((_EPW,), jnp.float32),
        pltpu.VMEM((_EPW,), jnp.float32),
        pltpu.VMEM((_EPW,), jnp.float32),
        pltpu.VMEM_SHARED((3 * _N_ATOMS,), jnp.float32),
        pltpu.SemaphoreType.DMA,
        pltpu.SemaphoreType.DMA,
        ],
    )(_sc_body)


def _vdw_body(lv_ref, s6_ref, e_ref, q_ref, v14_ref, c14_ref, ev_ref, ec_ref):
    sl = pl.ds(pl.program_id(0) * _BV, _BV)
    qscale = _CHARGE * _CHARGE / 100.0
    lv = lv_ref[...]
    r = 1.0 / lv
    r2 = r * r
    r6 = r2 * r2 * r2
    t = s6_ref[sl][None, :] * r6
    em = (e_ref[sl] * v14_ref[sl] * 0.01)[None, :]
    qm = (q_ref[sl] * c14_ref[sl] * qscale)[None, :]
    ev_ref[...] = em * (t * t - 2.0 * t)
    ec_ref[...] = qm * r


def _small_body(lb_ref, pb_ref, ta_ref, pa_ref, sc_ref, pt_ref, ci_ref, pi_ref,
                eb_ref, ea_ref, et_ref, ei_ref):
    db = lb_ref[...] - pb_ref[1:2, :]
    eb_ref[...] = (pb_ref[0:1, :] * 100.0) * db * db

    da = ta_ref[...] - pa_ref[1:2, :] * np.float32(np.pi / 10.0)
    ea_ref[...] = (pa_ref[0:1, :] * 10.0) * da * da

    et_ref[...] = (pt_ref[0:1, :] * sc_ref[:, 1, :]
                   + pt_ref[1:2, :] * sc_ref[:, 3, :]
                   + pt_ref[2:3, :] * sc_ref[:, 5, :]
                   + pt_ref[3:4, :] * sc_ref[:, 7, :])

    ei_ref[...] = pi_ref[...] * (1.0 - ci_ref[...])


_G = 4
_BV = _N_VDW // _G


def _row_spec(b):
    return pl.BlockSpec((16, b), lambda i: (0, i))


def _vec_spec(b):
    del b
    return pl.BlockSpec((_N_VDW,), lambda i: (0,))


_vdw_call = pl.pallas_call(
    _vdw_body,
    grid=(_G,),
    in_specs=[
        _row_spec(_BV), _vec_spec(_BV), _vec_spec(_BV), _vec_spec(_BV),
        _vec_spec(_BV), _vec_spec(_BV),
    ],
    out_specs=[_row_spec(_BV), _row_spec(_BV)],
    out_shape=[
        jax.ShapeDtypeStruct((16, _N_VDW), jnp.float32),
        jax.ShapeDtypeStruct((16, _N_VDW), jnp.float32),
    ],
)

_small_call = pl.pallas_call(
    _small_body,
    out_shape=[
        jax.ShapeDtypeStruct((16, 10000), jnp.float32),
        jax.ShapeDtypeStruct((16, 20000), jnp.float32),
        jax.ShapeDtypeStruct((16, 30000), jnp.float32),
        jax.ShapeDtypeStruct((16, 5000), jnp.float32),
    ],
)


def kernel(length_bond, theta_angle, length_vdw, non_bonded, vdw14, charge14,
           sin_cos_n_theta_torsion, cos2_imptors, paras_bond, paras_angle,
           paras_vdw, paras_charge, paras_torsion, paras_imptors):
    f32 = jnp.float32
    nb = non_bonded.astype(jnp.int32)

    tbl = jnp.concatenate(
        [paras_vdw[:, 0], paras_vdw[:, 1], paras_charge.astype(f32)])
    s6, e, q = _build_sc_gather()(nb[0], nb[1], tbl)

    E_bond, E_angle, E_torsion, E_imptors = _small_call(
        length_bond, paras_bond.T,
        theta_angle, paras_angle.T,
        jnp.transpose(sin_cos_n_theta_torsion, (0, 2, 1)), paras_torsion.T,
        cos2_imptors, paras_imptors.T,
    )

    E_vdw, E_charge = _vdw_call(length_vdw, s6, e, q, vdw14, charge14)

    E_ub = jnp.zeros((length_vdw.shape[0], 1), dtype=length_vdw.dtype)
    return (E_bond, E_angle, E_ub, E_vdw, E_charge, E_torsion, E_imptors)


# speedup vs baseline: 1.0214x; 1.0194x over previous
import functools

import jax
import jax.numpy as jnp
import numpy as np
from jax import lax
from jax.experimental import pallas as pl
from jax.experimental.pallas import tpu as pltpu
from jax.experimental.pallas import tpu_sc as plsc

_CHARGE = 18.222615
_N_ATOMS = 10000
_N_VDW = 320000

_NC = 2
_NS = 16
_L = 16
_NW = _NC * _NS
_EPW = _N_VDW // _NW
_SC_ITERS = _EPW // _L


def _sc_body(idx0_hbm, idx1_hbm, sig_hbm, eps_hbm, chg_hbm,
             s6_hbm, e_hbm, q_hbm,
             sig_v, eps_v, chg_v, i0_v, i1_v, s6_v, e_v, q_v,
             sig_sh, eps_sh, chg_sh, sem, bsem):
    sid = lax.axis_index("s")
    wid = sid * _NC + lax.axis_index("c")
    base = wid * _EPW
    cps = [
        pltpu.async_copy(idx0_hbm.at[pl.ds(base, _EPW)], i0_v, sem),
        pltpu.async_copy(idx1_hbm.at[pl.ds(base, _EPW)], i1_v, sem),
    ]
    @pl.when(sid == 0)
    def _():
        tc = [
            pltpu.async_copy(sig_hbm, sig_sh, bsem),
            pltpu.async_copy(eps_hbm, eps_sh, bsem),
            pltpu.async_copy(chg_hbm, chg_sh, bsem),
        ]
        for cp in tc:
            cp.wait()

    plsc.subcore_barrier()
    cps += [
        pltpu.async_copy(sig_sh, sig_v, sem),
        pltpu.async_copy(eps_sh, eps_v, sem),
        pltpu.async_copy(chg_sh, chg_v, sem),
    ]
    for cp in cps:
        cp.wait()

    def gather6(i0, i1):
        return (plsc.load_gather(sig_v, [i0]), plsc.load_gather(sig_v, [i1]),
                plsc.load_gather(eps_v, [i0]), plsc.load_gather(eps_v, [i1]),
                plsc.load_gather(chg_v, [i0]), plsc.load_gather(chg_v, [i1]))

    def emit(off, g):
        s1, s2, e1, e2, c1, c2 = g
        sg = s1 + s2
        sq = sg * sg
        s6_v[pl.ds(off, _L)] = sq * sq * sq
        e_v[pl.ds(off, _L)] = e1 * e2
        q_v[pl.ds(off, _L)] = c1 * c2

    def run_span(gs, n):
        g0 = gather6(i0_v[pl.ds(gs * _L, _L)], i1_v[pl.ds(gs * _L, _L)])
        nx = (gs + 1) * _L
        carry0 = (i0_v[pl.ds(nx, _L)], i1_v[pl.ds(nx, _L)]) + g0

        def body(i, carry):
            i0n, i1n = carry[0], carry[1]
            g = carry[2:]
            gn = gather6(i0n, i1n)
            off2 = jnp.minimum(gs + i + 2, _SC_ITERS - 1) * _L
            i0nn = i0_v[pl.ds(off2, _L)]
            i1nn = i1_v[pl.ds(off2, _L)]
            emit((gs + i) * _L, g)
            return (i0nn, i1nn) + gn

        last = lax.fori_loop(0, n - 1, body, carry0, unroll=4)
        emit((gs + n - 1) * _L, last[2:])

    _H1 = 313
    _E1 = _H1 * _L
    run_span(0, _H1)
    ocps = [
        pltpu.async_copy(s6_v.at[pl.ds(0, _E1)],
                         s6_hbm.at[pl.ds(base, _E1)], sem),
        pltpu.async_copy(e_v.at[pl.ds(0, _E1)],
                         e_hbm.at[pl.ds(base, _E1)], sem),
        pltpu.async_copy(q_v.at[pl.ds(0, _E1)],
                         q_hbm.at[pl.ds(base, _E1)], sem),
    ]
    run_span(_H1, _SC_ITERS - _H1)
    rem = _EPW - _E1
    ocps += [
        pltpu.async_copy(s6_v.at[pl.ds(_E1, rem)],
                         s6_hbm.at[pl.ds(base + _E1, rem)], sem),
        pltpu.async_copy(e_v.at[pl.ds(_E1, rem)],
                         e_hbm.at[pl.ds(base + _E1, rem)], sem),
        pltpu.async_copy(q_v.at[pl.ds(_E1, rem)],
                         q_hbm.at[pl.ds(base + _E1, rem)], sem),
    ]
    for cp in ocps:
        cp.wait()


@functools.lru_cache(maxsize=None)
def _build_sc_gather():
    return functools.partial(
        pl.kernel,
        mesh=plsc.VectorSubcoreMesh(core_axis_name="c", subcore_axis_name="s"),
        compiler_params=pltpu.CompilerParams(needs_layout_passes=False),
        out_type=[jax.ShapeDtypeStruct((_N_VDW,), jnp.float32)] * 3,
        scratch_types=[
        pltpu.VMEM((_N_ATOMS,), jnp.float32),
        pltpu.VMEM((_N_ATOMS,), jnp.float32),
        pltpu.VMEM((_N_ATOMS,), jnp.float32),
        pltpu.VMEM((_EPW,), jnp.int32),
        pltpu.VMEM((_EPW,), jnp.int32),
        pltpu.VMEM((_EPW,), jnp.float32),
        pltpu.VMEM((_EPW,), jnp.float32),
        pltpu.VMEM((_EPW,), jnp.float32),
        pltpu.VMEM_SHARED((_N_ATOMS,), jnp.float32),
        pltpu.VMEM_SHARED((_N_ATOMS,), jnp.float32),
        pltpu.VMEM_SHARED((_N_ATOMS,), jnp.float32),
        pltpu.SemaphoreType.DMA,
        pltpu.SemaphoreType.DMA,
        ],
    )(_sc_body)


def _vdw_body(lv_ref, s6_ref, e_ref, q_ref, v14_ref, c14_ref, ev_ref, ec_ref):
    sl = pl.ds(pl.program_id(0) * _BV, _BV)
    qscale = _CHARGE * _CHARGE / 100.0
    lv = lv_ref[...]
    r = 1.0 / lv
    r2 = r * r
    r6 = r2 * r2 * r2
    t = s6_ref[sl][None, :] * r6
    em = (e_ref[sl] * v14_ref[sl] * 0.01)[None, :]
    qm = (q_ref[sl] * c14_ref[sl] * qscale)[None, :]
    ev_ref[...] = em * (t * t - 2.0 * t)
    ec_ref[...] = qm * r


def _small_body(lb_ref, pb_ref, ta_ref, pa_ref, sc_ref, pt_ref, ci_ref, pi_ref,
                eb_ref, ea_ref, et_ref, ei_ref):
    db = lb_ref[...] - pb_ref[1:2, :]
    eb_ref[...] = (pb_ref[0:1, :] * 100.0) * db * db

    da = ta_ref[...] - pa_ref[1:2, :] * np.float32(np.pi / 10.0)
    ea_ref[...] = (pa_ref[0:1, :] * 10.0) * da * da

    et_ref[...] = (pt_ref[0:1, :] * sc_ref[:, 1, :]
                   + pt_ref[1:2, :] * sc_ref[:, 3, :]
                   + pt_ref[2:3, :] * sc_ref[:, 5, :]
                   + pt_ref[3:4, :] * sc_ref[:, 7, :])

    ei_ref[...] = pi_ref[...] * (1.0 - ci_ref[...])


_G = 4
_BV = _N_VDW // _G


def _row_spec(b):
    return pl.BlockSpec((16, b), lambda i: (0, i))


def _vec_spec(b):
    del b
    return pl.BlockSpec((_N_VDW,), lambda i: (0,))


_vdw_call = pl.pallas_call(
    _vdw_body,
    grid=(_G,),
    in_specs=[
        _row_spec(_BV), _vec_spec(_BV), _vec_spec(_BV), _vec_spec(_BV),
        _vec_spec(_BV), _vec_spec(_BV),
    ],
    out_specs=[_row_spec(_BV), _row_spec(_BV)],
    out_shape=[
        jax.ShapeDtypeStruct((16, _N_VDW), jnp.float32),
        jax.ShapeDtypeStruct((16, _N_VDW), jnp.float32),
    ],
)

_small_call = pl.pallas_call(
    _small_body,
    out_shape=[
        jax.ShapeDtypeStruct((16, 10000), jnp.float32),
        jax.ShapeDtypeStruct((16, 20000), jnp.float32),
        jax.ShapeDtypeStruct((16, 30000), jnp.float32),
        jax.ShapeDtypeStruct((16, 5000), jnp.float32),
    ],
)


def kernel(length_bond, theta_angle, length_vdw, non_bonded, vdw14, charge14,
           sin_cos_n_theta_torsion, cos2_imptors, paras_bond, paras_angle,
           paras_vdw, paras_charge, paras_torsion, paras_imptors):
    f32 = jnp.float32
    nb = non_bonded.astype(jnp.int32)

    s6, e, q = _build_sc_gather()(
        nb[0], nb[1],
        paras_vdw[:, 0], paras_vdw[:, 1], paras_charge.astype(f32))

    E_bond, E_angle, E_torsion, E_imptors = _small_call(
        length_bond, paras_bond.T,
        theta_angle, paras_angle.T,
        jnp.transpose(sin_cos_n_theta_torsion, (0, 2, 1)), paras_torsion.T,
        cos2_imptors, paras_imptors.T,
    )

    E_vdw, E_charge = _vdw_call(length_vdw, s6, e, q, vdw14, charge14)

    E_ub = jnp.zeros((length_vdw.shape[0], 1), dtype=length_vdw.dtype)
    return (E_bond, E_angle, E_ub, E_vdw, E_charge, E_torsion, E_imptors)


# speedup vs baseline: 1.0228x; 1.0014x over previous
import functools

import jax
import jax.numpy as jnp
import numpy as np
from jax import lax
from jax.experimental import pallas as pl
from jax.experimental.pallas import tpu as pltpu
from jax.experimental.pallas import tpu_sc as plsc

_CHARGE = 18.222615
_N_ATOMS = 10000
_N_VDW = 320000

_NC = 2
_NS = 16
_L = 16
_NW = _NC * _NS
_EPW = _N_VDW // _NW
_SC_ITERS = _EPW // _L


def _sc_body(idx0_hbm, idx1_hbm, sig_hbm, eps_hbm, chg_hbm,
             s6_hbm, e_hbm, q_hbm,
             sig_v, eps_v, chg_v, i0_v, i1_v, s6_v, e_v, q_v,
             sig_sh, eps_sh, chg_sh, sem, bsem):
    sid = lax.axis_index("s")
    wid = sid * _NC + lax.axis_index("c")
    base = wid * _EPW
    cps = [
        pltpu.async_copy(idx0_hbm.at[pl.ds(base, _EPW)], i0_v, sem),
        pltpu.async_copy(idx1_hbm.at[pl.ds(base, _EPW)], i1_v, sem),
    ]
    @pl.when(sid == 0)
    def _():
        tc = [
            pltpu.async_copy(sig_hbm, sig_sh, bsem),
            pltpu.async_copy(eps_hbm, eps_sh, bsem),
            pltpu.async_copy(chg_hbm, chg_sh, bsem),
        ]
        for cp in tc:
            cp.wait()

    plsc.subcore_barrier()
    cps += [
        pltpu.async_copy(sig_sh, sig_v, sem),
        pltpu.async_copy(eps_sh, eps_v, sem),
        pltpu.async_copy(chg_sh, chg_v, sem),
    ]
    for cp in cps:
        cp.wait()

    def gather6(i0, i1):
        return (plsc.load_gather(sig_v, [i0]), plsc.load_gather(sig_v, [i1]),
                plsc.load_gather(eps_v, [i0]), plsc.load_gather(eps_v, [i1]),
                plsc.load_gather(chg_v, [i0]), plsc.load_gather(chg_v, [i1]))

    def emit(off, g):
        s1, s2, e1, e2, c1, c2 = g
        sg = s1 + s2
        sq = sg * sg
        s6_v[pl.ds(off, _L)] = sq * sq * sq
        e_v[pl.ds(off, _L)] = e1 * e2
        q_v[pl.ds(off, _L)] = c1 * c2

    def run_span(gs, n):
        g0 = gather6(i0_v[pl.ds(gs * _L, _L)], i1_v[pl.ds(gs * _L, _L)])
        nx = (gs + 1) * _L
        carry0 = (i0_v[pl.ds(nx, _L)], i1_v[pl.ds(nx, _L)]) + g0

        def body(i, carry):
            i0n, i1n = carry[0], carry[1]
            g = carry[2:]
            gn = gather6(i0n, i1n)
            off2 = jnp.minimum(gs + i + 2, _SC_ITERS - 1) * _L
            i0nn = i0_v[pl.ds(off2, _L)]
            i1nn = i1_v[pl.ds(off2, _L)]
            emit((gs + i) * _L, g)
            return (i0nn, i1nn) + gn

        last = lax.fori_loop(0, n - 1, body, carry0, unroll=8)
        emit((gs + n - 1) * _L, last[2:])

    _H1 = 313
    _E1 = _H1 * _L
    run_span(0, _H1)
    ocps = [
        pltpu.async_copy(s6_v.at[pl.ds(0, _E1)],
                         s6_hbm.at[pl.ds(base, _E1)], sem),
        pltpu.async_copy(e_v.at[pl.ds(0, _E1)],
                         e_hbm.at[pl.ds(base, _E1)], sem),
        pltpu.async_copy(q_v.at[pl.ds(0, _E1)],
                         q_hbm.at[pl.ds(base, _E1)], sem),
    ]
    run_span(_H1, _SC_ITERS - _H1)
    rem = _EPW - _E1
    ocps += [
        pltpu.async_copy(s6_v.at[pl.ds(_E1, rem)],
                         s6_hbm.at[pl.ds(base + _E1, rem)], sem),
        pltpu.async_copy(e_v.at[pl.ds(_E1, rem)],
                         e_hbm.at[pl.ds(base + _E1, rem)], sem),
        pltpu.async_copy(q_v.at[pl.ds(_E1, rem)],
                         q_hbm.at[pl.ds(base + _E1, rem)], sem),
    ]
    for cp in ocps:
        cp.wait()


@functools.lru_cache(maxsize=None)
def _build_sc_gather():
    return functools.partial(
        pl.kernel,
        mesh=plsc.VectorSubcoreMesh(core_axis_name="c", subcore_axis_name="s"),
        compiler_params=pltpu.CompilerParams(needs_layout_passes=False),
        out_type=[jax.ShapeDtypeStruct((_N_VDW,), jnp.float32)] * 3,
        scratch_types=[
        pltpu.VMEM((_N_ATOMS,), jnp.float32),
        pltpu.VMEM((_N_ATOMS,), jnp.float32),
        pltpu.VMEM((_N_ATOMS,), jnp.float32),
        pltpu.VMEM((_EPW,), jnp.int32),
        pltpu.VMEM((_EPW,), jnp.int32),
        pltpu.VMEM((_EPW,), jnp.float32),
        pltpu.VMEM((_EPW,), jnp.float32),
        pltpu.VMEM((_EPW,), jnp.float32),
        pltpu.VMEM_SHARED((_N_ATOMS,), jnp.float32),
        pltpu.VMEM_SHARED((_N_ATOMS,), jnp.float32),
        pltpu.VMEM_SHARED((_N_ATOMS,), jnp.float32),
        pltpu.SemaphoreType.DMA,
        pltpu.SemaphoreType.DMA,
        ],
    )(_sc_body)


def _vdw_body(lv_ref, s6_ref, e_ref, q_ref, v14_ref, c14_ref, ev_ref, ec_ref):
    sl = pl.ds(pl.program_id(0) * _BV, _BV)
    qscale = _CHARGE * _CHARGE / 100.0
    lv = lv_ref[...]
    r = 1.0 / lv
    r2 = r * r
    r6 = r2 * r2 * r2
    t = s6_ref[sl][None, :] * r6
    em = (e_ref[sl] * v14_ref[sl] * 0.01)[None, :]
    qm = (q_ref[sl] * c14_ref[sl] * qscale)[None, :]
    ev_ref[...] = em * (t * t - 2.0 * t)
    ec_ref[...] = qm * r


def _small_body(lb_ref, pb_ref, ta_ref, pa_ref, sc_ref, pt_ref, ci_ref, pi_ref,
                eb_ref, ea_ref, et_ref, ei_ref):
    db = lb_ref[...] - pb_ref[1:2, :]
    eb_ref[...] = (pb_ref[0:1, :] * 100.0) * db * db

    da = ta_ref[...] - pa_ref[1:2, :] * np.float32(np.pi / 10.0)
    ea_ref[...] = (pa_ref[0:1, :] * 10.0) * da * da

    et_ref[...] = (pt_ref[0:1, :] * sc_ref[:, 1, :]
                   + pt_ref[1:2, :] * sc_ref[:, 3, :]
                   + pt_ref[2:3, :] * sc_ref[:, 5, :]
                   + pt_ref[3:4, :] * sc_ref[:, 7, :])

    ei_ref[...] = pi_ref[...] * (1.0 - ci_ref[...])


_G = 4
_BV = _N_VDW // _G


def _row_spec(b):
    return pl.BlockSpec((16, b), lambda i: (0, i))


def _vec_spec(b):
    del b
    return pl.BlockSpec((_N_VDW,), lambda i: (0,))


_vdw_call = pl.pallas_call(
    _vdw_body,
    grid=(_G,),
    in_specs=[
        _row_spec(_BV), _vec_spec(_BV), _vec_spec(_BV), _vec_spec(_BV),
        _vec_spec(_BV), _vec_spec(_BV),
    ],
    out_specs=[_row_spec(_BV), _row_spec(_BV)],
    out_shape=[
        jax.ShapeDtypeStruct((16, _N_VDW), jnp.float32),
        jax.ShapeDtypeStruct((16, _N_VDW), jnp.float32),
    ],
)

_small_call = pl.pallas_call(
    _small_body,
    out_shape=[
        jax.ShapeDtypeStruct((16, 10000), jnp.float32),
        jax.ShapeDtypeStruct((16, 20000), jnp.float32),
        jax.ShapeDtypeStruct((16, 30000), jnp.float32),
        jax.ShapeDtypeStruct((16, 5000), jnp.float32),
    ],
)


def kernel(length_bond, theta_angle, length_vdw, non_bonded, vdw14, charge14,
           sin_cos_n_theta_torsion, cos2_imptors, paras_bond, paras_angle,
           paras_vdw, paras_charge, paras_torsion, paras_imptors):
    f32 = jnp.float32
    nb = non_bonded.astype(jnp.int32)

    s6, e, q = _build_sc_gather()(
        nb[0], nb[1],
        paras_vdw[:, 0], paras_vdw[:, 1], paras_charge.astype(f32))

    E_bond, E_angle, E_torsion, E_imptors = _small_call(
        length_bond, paras_bond.T,
        theta_angle, paras_angle.T,
        jnp.transpose(sin_cos_n_theta_torsion, (0, 2, 1)), paras_torsion.T,
        cos2_imptors, paras_imptors.T,
    )

    E_vdw, E_charge = _vdw_call(length_vdw, s6, e, q, vdw14, charge14)

    E_ub = jnp.zeros((length_vdw.shape[0], 1), dtype=length_vdw.dtype)
    return (E_bond, E_angle, E_ub, E_vdw, E_charge, E_torsion, E_imptors)
